# baseline (device time: 169627 ns/iter reference)
import jax
import jax.numpy as jnp
from jax import lax
from jax.experimental import pallas as pl
from jax.experimental.pallas import tpu as pltpu

N_DEV = 4
B_PER = 2
HQ_PER = 8
SQ = 512
SKV = 512
DH = 64
DMODEL = 768
HDIM = HQ_PER * DH
BLK = 64
N_PHASE = N_DEV * B_PER

_CompilerParams = getattr(pltpu, "CompilerParams", None) or getattr(
    pltpu, "TPUCompilerParams"
)
_ANY = pl.ANY


def kernel(x, Wq, K_ext, V_ext, Wo):
    bf16 = jnp.bfloat16
    x_bf = x.astype(bf16)
    payload = jnp.stack([Wq.astype(bf16), Wo.T.astype(bf16)])

    def body(
        x_ref, k_hbm, v_hbm, pay_ref, out_ref,
        comm_ref, q_ref, ctx_ref, bias_ref, kbuf, vbuf,
        send_sems, recv_sems, dma_sems,
    ):
        my_i = lax.axis_index("i")
        left = lax.rem(my_i + N_DEV - 1, N_DEV)
        right = lax.rem(my_i + 1, N_DEV)
        origins = [my_i, left, right, lax.rem(my_i + 2, N_DEV)]

        def issue_phase(phase, buf):
            slot, b = divmod(phase, B_PER)
            bg = my_i * B_PER + b
            hb = origins[slot] * HQ_PER
            dmas = []
            for h in range(HQ_PER):
                for src, dst in ((k_hbm, kbuf), (v_hbm, vbuf)):
                    d = pltpu.make_async_copy(
                        src.at[bg, :, hb + h, :],
                        dst.at[buf, h],
                        dma_sems.at[buf],
                    )
                    d.start()
                    dmas.append(d)
            return dmas

        pending = issue_phase(0, 0)

        row = lax.broadcasted_iota(jnp.int32, (SQ, SKV), 0)
        col = lax.broadcasted_iota(jnp.int32, (SQ, SKV), 1)
        bias_ref[...] = jnp.where(
            col // BLK <= row // BLK, 0.0, -1e9
        ).astype(jnp.float32)

        comm_ref[0] = pay_ref[...]

        barrier_sem = pltpu.get_barrier_semaphore()
        for nbr in (left, right):
            pl.semaphore_signal(
                barrier_sem, inc=1,
                device_id=(nbr,), device_id_type=pl.DeviceIdType.MESH,
            )
        pl.semaphore_wait(barrier_sem, 2)

        def remote_copy(src, dst, sem_idx, dev):
            return pltpu.make_async_remote_copy(
                src_ref=src, dst_ref=dst,
                send_sem=send_sems.at[sem_idx],
                recv_sem=recv_sems.at[sem_idx],
                device_id=(dev,),
                device_id_type=pl.DeviceIdType.MESH,
            )

        rdma_a = remote_copy(comm_ref.at[0], comm_ref.at[1], 0, right)
        rdma_b = remote_copy(comm_ref.at[0], comm_ref.at[2], 1, left)
        rdma_a.start()
        rdma_b.start()
        rdma_c = remote_copy(comm_ref.at[1, 0], comm_ref.at[3, 0], 2, right)
        rdma_d = remote_copy(comm_ref.at[2, 1], comm_ref.at[3, 1], 3, left)

        def compute_batch(slot, b, buf):
            hbase = origins[slot] * HQ_PER
            wq = comm_ref[slot, 0]
            woT = comm_ref[slot, 1]
            q_ref[...] = lax.dot_general(
                x_ref[b], wq, (((1,), (0,)), ((), ())),
                preferred_element_type=jnp.float32,
            ).astype(jnp.bfloat16)
            for h in range(HQ_PER):
                k_h = kbuf[buf, h].astype(jnp.bfloat16)
                v_h = vbuf[buf, h].astype(jnp.bfloat16)
                for lo, hi, nk in ((0, SQ // 2, SKV // 2),
                                   (SQ // 2, SQ, SKV)):
                    scores = lax.dot_general(
                        q_ref[lo:hi, h * DH:(h + 1) * DH], k_h[:nk],
                        (((1,), (1,)), ((), ())),
                        preferred_element_type=jnp.float32,
                    )
                    e = jnp.exp(
                        scores * 0.125 + bias_ref[lo:hi, :nk]
                    ).astype(jnp.bfloat16)
                    s = lax.dot_general(
                        e, v_h[:nk], (((1,), (0,)), ((), ())),
                        preferred_element_type=jnp.float32,
                    )
                    denom = jnp.sum(
                        e.astype(jnp.float32), axis=-1, keepdims=True
                    )
                    ctx_ref[lo:hi, h * DH:(h + 1) * DH] = (
                        s / denom
                    ).astype(jnp.bfloat16)
            part = lax.dot_general(
                ctx_ref[...], woT, (((1,), (1,)), ((), ())),
                preferred_element_type=jnp.float32,
            )
            if slot == 0:
                out_ref[b] = part
            else:
                out_ref[b] = out_ref[b] + part

        for phase in range(N_PHASE):
            slot, b = divmod(phase, B_PER)
            buf = phase % 2
            if phase == 2:
                rdma_a.wait_recv()
                rdma_b.wait_recv()
                rdma_c.start()
                rdma_d.start()
            if phase == 6:
                rdma_c.wait_recv()
                rdma_d.wait_recv()
            nxt = issue_phase(phase + 1, 1 - buf) if phase < N_PHASE - 1 else []
            for d in pending:
                d.wait()
            compute_batch(slot, b, buf)
            pending = nxt

        rdma_a.wait_send()
        rdma_b.wait_send()
        rdma_c.wait_send()
        rdma_d.wait_send()

    return pl.pallas_call(
        body,
        out_shape=jax.ShapeDtypeStruct((B_PER, SQ, DMODEL), jnp.float32),
        in_specs=[
            pl.BlockSpec(memory_space=pltpu.VMEM),
            pl.BlockSpec(memory_space=_ANY),
            pl.BlockSpec(memory_space=_ANY),
            pl.BlockSpec(memory_space=pltpu.VMEM),
        ],
        out_specs=pl.BlockSpec(memory_space=pltpu.VMEM),
        scratch_shapes=[
            pltpu.VMEM((N_DEV, 2, DMODEL, HDIM), bf16),
            pltpu.VMEM((SQ, HDIM), bf16),
            pltpu.VMEM((SQ, HDIM), bf16),
            pltpu.VMEM((SQ, SKV), jnp.float32),
            pltpu.VMEM((2, HQ_PER, SKV, DH), jnp.float32),
            pltpu.VMEM((2, HQ_PER, SKV, DH), jnp.float32),
            pltpu.SemaphoreType.DMA((4,)),
            pltpu.SemaphoreType.DMA((4,)),
            pltpu.SemaphoreType.DMA((2,)),
        ],
        compiler_params=_CompilerParams(collective_id=0),
    )(x_bf, K_ext, V_ext, payload)


# device time: 118946 ns/iter; 1.4261x vs baseline; 1.4261x over previous
import jax
import jax.numpy as jnp
from jax import lax
from jax.experimental import pallas as pl
from jax.experimental.pallas import tpu as pltpu

N_DEV = 4
B_PER = 2
HQ_PER = 8
SQ = 512
SKV = 512
DH = 64
DMODEL = 768
HDIM = HQ_PER * DH
KVW = 32 * DH
BLK = 64
N_PHASE = N_DEV * B_PER

_CompilerParams = getattr(pltpu, "CompilerParams", None) or getattr(
    pltpu, "TPUCompilerParams"
)


def kernel(x, Wq, K_ext, V_ext, Wo):
    bf16 = jnp.bfloat16
    x_bf = x.astype(bf16)
    K_r = K_ext.reshape(8, SKV, KVW)
    V_r = V_ext.reshape(8, SKV, KVW)
    payload = jnp.stack([Wq.astype(bf16), Wo.T.astype(bf16)])

    def body(
        x_ref, k_hbm, v_hbm, pay_ref, out_ref,
        comm_ref, q_ref, ctx_ref, bias_ref, kstage, vstage, kblk, vblk,
        send_sems, recv_sems, stage_sems, blk_sems,
    ):
        my_i = lax.axis_index("i")
        left = lax.rem(my_i + N_DEV - 1, N_DEV)
        right = lax.rem(my_i + 1, N_DEV)
        origins = [my_i, left, right, lax.rem(my_i + 2, N_DEV)]

        stage1 = []
        for b in range(B_PER):
            bg = my_i * B_PER + b
            for src, dst in ((k_hbm, kstage), (v_hbm, vstage)):
                d = pltpu.make_async_copy(
                    src.at[bg], dst.at[b], stage_sems.at[b]
                )
                d.start()
                stage1.append(d)

        row = lax.broadcasted_iota(jnp.int32, (SQ, SKV), 0)
        col = lax.broadcasted_iota(jnp.int32, (SQ, SKV), 1)
        bias_ref[...] = jnp.where(
            col // BLK <= row // BLK, 0.0, -1e9
        ).astype(jnp.float32)

        comm_ref[0] = pay_ref[...]

        barrier_sem = pltpu.get_barrier_semaphore()
        for nbr in (left, right):
            pl.semaphore_signal(
                barrier_sem, inc=1,
                device_id=(nbr,), device_id_type=pl.DeviceIdType.MESH,
            )
        pl.semaphore_wait(barrier_sem, 2)

        def remote_copy(src, dst, sem_idx, dev):
            return pltpu.make_async_remote_copy(
                src_ref=src, dst_ref=dst,
                send_sem=send_sems.at[sem_idx],
                recv_sem=recv_sems.at[sem_idx],
                device_id=(dev,),
                device_id_type=pl.DeviceIdType.MESH,
            )

        rdma_a = remote_copy(comm_ref.at[0], comm_ref.at[1], 0, right)
        rdma_b = remote_copy(comm_ref.at[0], comm_ref.at[2], 1, left)
        rdma_a.start()
        rdma_b.start()
        rdma_c = remote_copy(comm_ref.at[1, 0], comm_ref.at[3, 0], 2, right)
        rdma_d = remote_copy(comm_ref.at[2, 1], comm_ref.at[3, 1], 3, left)

        for d in stage1:
            d.wait()

        def issue_blk(phase, buf):
            slot, b = divmod(phase, B_PER)
            off = origins[slot] * HDIM
            dmas = []
            for src, dst in ((kstage, kblk), (vstage, vblk)):
                d = pltpu.make_async_copy(
                    src.at[b, :, pl.ds(off, HDIM)],
                    dst.at[buf],
                    blk_sems.at[buf],
                )
                d.start()
                dmas.append(d)
            return dmas

        pending = issue_blk(0, 0)

        def compute_batch(slot, b, buf):
            wq = comm_ref[slot, 0]
            woT = comm_ref[slot, 1]
            q_ref[...] = lax.dot_general(
                x_ref[b], wq, (((1,), (0,)), ((), ())),
                preferred_element_type=jnp.float32,
            ).astype(jnp.bfloat16)
            for h in range(HQ_PER):
                k_h = kblk[buf, :, h * DH:(h + 1) * DH].astype(jnp.bfloat16)
                v_h = vblk[buf, :, h * DH:(h + 1) * DH].astype(jnp.bfloat16)
                for lo, hi, nk in ((0, SQ // 2, SKV // 2),
                                   (SQ // 2, SQ, SKV)):
                    scores = lax.dot_general(
                        q_ref[lo:hi, h * DH:(h + 1) * DH], k_h[:nk],
                        (((1,), (1,)), ((), ())),
                        preferred_element_type=jnp.float32,
                    )
                    e = jnp.exp(
                        scores * 0.125 + bias_ref[lo:hi, :nk]
                    ).astype(jnp.bfloat16)
                    s = lax.dot_general(
                        e, v_h[:nk], (((1,), (0,)), ((), ())),
                        preferred_element_type=jnp.float32,
                    )
                    denom = jnp.sum(
                        e.astype(jnp.float32), axis=-1, keepdims=True
                    )
                    ctx_ref[lo:hi, h * DH:(h + 1) * DH] = (
                        s / denom
                    ).astype(jnp.bfloat16)
            part = lax.dot_general(
                ctx_ref[...], woT, (((1,), (1,)), ((), ())),
                preferred_element_type=jnp.float32,
            )
            if slot == 0:
                out_ref[b] = part
            else:
                out_ref[b] = out_ref[b] + part

        for phase in range(N_PHASE):
            slot, b = divmod(phase, B_PER)
            buf = phase % 2
            if phase == 2:
                rdma_a.wait_recv()
                rdma_b.wait_recv()
                rdma_c.start()
                rdma_d.start()
            if phase == 6:
                rdma_c.wait_recv()
                rdma_d.wait_recv()
            nxt = issue_blk(phase + 1, 1 - buf) if phase < N_PHASE - 1 else []
            for d in pending:
                d.wait()
            compute_batch(slot, b, buf)
            pending = nxt

        rdma_a.wait_send()
        rdma_b.wait_send()
        rdma_c.wait_send()
        rdma_d.wait_send()

    return pl.pallas_call(
        body,
        out_shape=jax.ShapeDtypeStruct((B_PER, SQ, DMODEL), jnp.float32),
        in_specs=[
            pl.BlockSpec(memory_space=pltpu.VMEM),
            pl.BlockSpec(memory_space=pl.ANY),
            pl.BlockSpec(memory_space=pl.ANY),
            pl.BlockSpec(memory_space=pltpu.VMEM),
        ],
        out_specs=pl.BlockSpec(memory_space=pltpu.VMEM),
        scratch_shapes=[
            pltpu.VMEM((N_DEV, 2, DMODEL, HDIM), bf16),
            pltpu.VMEM((SQ, HDIM), bf16),
            pltpu.VMEM((SQ, HDIM), bf16),
            pltpu.VMEM((SQ, SKV), jnp.float32),
            pltpu.VMEM((B_PER, SKV, KVW), jnp.float32),
            pltpu.VMEM((B_PER, SKV, KVW), jnp.float32),
            pltpu.VMEM((2, SKV, HDIM), jnp.float32),
            pltpu.VMEM((2, SKV, HDIM), jnp.float32),
            pltpu.SemaphoreType.DMA((4,)),
            pltpu.SemaphoreType.DMA((4,)),
            pltpu.SemaphoreType.DMA((B_PER,)),
            pltpu.SemaphoreType.DMA((2,)),
        ],
        compiler_params=_CompilerParams(collective_id=0),
    )(x_bf, K_r, V_r, payload)


# device time: 66034 ns/iter; 2.5688x vs baseline; 1.8013x over previous
import jax
import jax.numpy as jnp
from jax import lax
from jax.experimental import pallas as pl
from jax.experimental.pallas import tpu as pltpu

N_DEV = 4
B_PER = 2
HQ_PER = 8
SQ = 512
SKV = 512
DH = 64
DMODEL = 768
HDIM = HQ_PER * DH
KVW = 32 * DH
BLK = 64
N_PHASE = N_DEV * B_PER

_CompilerParams = getattr(pltpu, "CompilerParams", None) or getattr(
    pltpu, "TPUCompilerParams"
)


def kernel(x, Wq, K_ext, V_ext, Wo):
    bf16 = jnp.bfloat16
    my = lax.axis_index("i")
    x_bf = x.astype(bf16)

    orig = [my, (my + 3) % N_DEV, (my + 1) % N_DEV, (my + 2) % N_DEV]

    def prep(t):
        t = lax.dynamic_slice_in_dim(
            t.reshape(8, SKV, KVW), my * B_PER, B_PER, axis=0
        ).astype(bf16)
        return jnp.concatenate(
            [
                lax.dynamic_slice_in_dim(t, o * HDIM, HDIM, axis=2)
                for o in orig
            ],
            axis=2,
        )

    K_ord = prep(K_ext)
    V_ord = prep(V_ext)
    payload = jnp.stack([Wq.astype(bf16), Wo.T.astype(bf16)])

    def body(
        x_ref, k_ref, v_ref, pay_ref, out_ref,
        comm_ref, q_ref, ctx_ref, bias_ref,
        send_sems, recv_sems,
    ):
        my_i = lax.axis_index("i")
        left = lax.rem(my_i + N_DEV - 1, N_DEV)
        right = lax.rem(my_i + 1, N_DEV)

        row = lax.broadcasted_iota(jnp.int32, (SQ, SKV), 0)
        col = lax.broadcasted_iota(jnp.int32, (SQ, SKV), 1)
        bias_ref[...] = jnp.where(
            col // BLK <= row // BLK, 0.0, -1e9
        ).astype(jnp.float32)

        comm_ref[0] = pay_ref[...]

        barrier_sem = pltpu.get_barrier_semaphore()
        for nbr in (left, right):
            pl.semaphore_signal(
                barrier_sem, inc=1,
                device_id=(nbr,), device_id_type=pl.DeviceIdType.MESH,
            )
        pl.semaphore_wait(barrier_sem, 2)

        def remote_copy(src, dst, sem_idx, dev):
            return pltpu.make_async_remote_copy(
                src_ref=src, dst_ref=dst,
                send_sem=send_sems.at[sem_idx],
                recv_sem=recv_sems.at[sem_idx],
                device_id=(dev,),
                device_id_type=pl.DeviceIdType.MESH,
            )

        rdma_a = remote_copy(comm_ref.at[0], comm_ref.at[1], 0, right)
        rdma_b = remote_copy(comm_ref.at[0], comm_ref.at[2], 1, left)
        rdma_a.start()
        rdma_b.start()
        rdma_c = remote_copy(comm_ref.at[1, 0], comm_ref.at[3, 0], 2, right)
        rdma_d = remote_copy(comm_ref.at[2, 1], comm_ref.at[3, 1], 3, left)

        def compute_batch(slot, b):
            wq = comm_ref[slot, 0]
            woT = comm_ref[slot, 1]
            q_ref[...] = lax.dot_general(
                x_ref[b], wq, (((1,), (0,)), ((), ())),
                preferred_element_type=jnp.float32,
            ).astype(jnp.bfloat16)
            base = slot * HDIM
            for h in range(HQ_PER):
                cols = slice(base + h * DH, base + (h + 1) * DH)
                k_h = k_ref[b, :, cols]
                v_h = v_ref[b, :, cols]
                for lo, hi, nk in ((0, SQ // 2, SKV // 2),
                                   (SQ // 2, SQ, SKV)):
                    scores = lax.dot_general(
                        q_ref[lo:hi, h * DH:(h + 1) * DH], k_h[:nk],
                        (((1,), (1,)), ((), ())),
                        preferred_element_type=jnp.float32,
                    )
                    e = jnp.exp(
                        scores * 0.125 + bias_ref[lo:hi, :nk]
                    ).astype(jnp.bfloat16)
                    s = lax.dot_general(
                        e, v_h[:nk], (((1,), (0,)), ((), ())),
                        preferred_element_type=jnp.float32,
                    )
                    denom = jnp.sum(
                        e.astype(jnp.float32), axis=-1, keepdims=True
                    )
                    ctx_ref[lo:hi, h * DH:(h + 1) * DH] = (
                        s / denom
                    ).astype(jnp.bfloat16)
            part = lax.dot_general(
                ctx_ref[...], woT, (((1,), (1,)), ((), ())),
                preferred_element_type=jnp.float32,
            )
            if slot == 0:
                out_ref[b] = part
            else:
                out_ref[b] = out_ref[b] + part

        for phase in range(N_PHASE):
            slot, b = divmod(phase, B_PER)
            if phase == 2:
                rdma_a.wait_recv()
                rdma_b.wait_recv()
                rdma_c.start()
                rdma_d.start()
            if phase == 6:
                rdma_c.wait_recv()
                rdma_d.wait_recv()
            compute_batch(slot, b)

        rdma_a.wait_send()
        rdma_b.wait_send()
        rdma_c.wait_send()
        rdma_d.wait_send()

    return pl.pallas_call(
        body,
        out_shape=jax.ShapeDtypeStruct((B_PER, SQ, DMODEL), jnp.float32),
        in_specs=[pl.BlockSpec(memory_space=pltpu.VMEM)] * 4,
        out_specs=pl.BlockSpec(memory_space=pltpu.VMEM),
        scratch_shapes=[
            pltpu.VMEM((N_DEV, 2, DMODEL, HDIM), bf16),
            pltpu.VMEM((SQ, HDIM), bf16),
            pltpu.VMEM((SQ, HDIM), bf16),
            pltpu.VMEM((SQ, SKV), jnp.float32),
            pltpu.SemaphoreType.DMA((4,)),
            pltpu.SemaphoreType.DMA((4,)),
        ],
        compiler_params=_CompilerParams(collective_id=0),
    )(x_bf, K_ord, V_ord, payload)


# device time: 62235 ns/iter; 2.7256x vs baseline; 1.0610x over previous
import jax
import jax.numpy as jnp
from jax import lax
from jax.experimental import pallas as pl
from jax.experimental.pallas import tpu as pltpu

N_DEV = 4
B_PER = 2
HQ_PER = 8
SQ = 512
SKV = 512
DH = 64
DMODEL = 768
HDIM = HQ_PER * DH
KVW = 32 * DH
BLK = 64
N_PHASE = N_DEV * B_PER

_CompilerParams = getattr(pltpu, "CompilerParams", None) or getattr(
    pltpu, "TPUCompilerParams"
)


def kernel(x, Wq, K_ext, V_ext, Wo):
    bf16 = jnp.bfloat16
    my = lax.axis_index("i")
    x_bf = x.astype(bf16)

    def prep(t):
        return lax.dynamic_slice_in_dim(
            t.reshape(8, SKV, KVW), my * B_PER, B_PER, axis=0
        ).astype(bf16)

    K_ord = prep(K_ext)
    V_ord = prep(V_ext)
    payload = jnp.stack([Wq.astype(bf16), Wo.T.astype(bf16)])

    def body(
        x_ref, k_ref, v_ref, pay_ref, out_ref,
        comm_ref, q_ref, ctx_ref, bias_ref, kblk, vblk,
        send_sems, recv_sems, blk_sems,
    ):
        my_i = lax.axis_index("i")
        left = lax.rem(my_i + N_DEV - 1, N_DEV)
        right = lax.rem(my_i + 1, N_DEV)
        origins = [my_i, left, right, lax.rem(my_i + 2, N_DEV)]

        def issue_blk(phase, buf):
            slot, b = divmod(phase, B_PER)
            off = origins[slot] * HDIM
            dmas = []
            for src, dst in ((k_ref, kblk), (v_ref, vblk)):
                d = pltpu.make_async_copy(
                    src.at[b, :, pl.ds(off, HDIM)],
                    dst.at[buf],
                    blk_sems.at[buf],
                )
                d.start()
                dmas.append(d)
            return dmas

        pending = issue_blk(0, 0)

        row = lax.broadcasted_iota(jnp.int32, (SQ, SKV), 0)
        col = lax.broadcasted_iota(jnp.int32, (SQ, SKV), 1)
        bias_ref[...] = jnp.where(
            col // BLK <= row // BLK, 0.0, -1e9
        ).astype(jnp.float32)

        comm_ref[0] = pay_ref[...]

        barrier_sem = pltpu.get_barrier_semaphore()
        for nbr in (left, right):
            pl.semaphore_signal(
                barrier_sem, inc=1,
                device_id=(nbr,), device_id_type=pl.DeviceIdType.MESH,
            )
        pl.semaphore_wait(barrier_sem, 2)

        def remote_copy(src, dst, sem_idx, dev):
            return pltpu.make_async_remote_copy(
                src_ref=src, dst_ref=dst,
                send_sem=send_sems.at[sem_idx],
                recv_sem=recv_sems.at[sem_idx],
                device_id=(dev,),
                device_id_type=pl.DeviceIdType.MESH,
            )

        rdma_a = remote_copy(comm_ref.at[0], comm_ref.at[1], 0, right)
        rdma_b = remote_copy(comm_ref.at[0], comm_ref.at[2], 1, left)
        rdma_a.start()
        rdma_b.start()
        rdma_c = remote_copy(comm_ref.at[1, 0], comm_ref.at[3, 0], 2, right)
        rdma_d = remote_copy(comm_ref.at[2, 1], comm_ref.at[3, 1], 3, left)

        def compute_batch(slot, b, buf):
            wq = comm_ref[slot, 0]
            woT = comm_ref[slot, 1]
            q_ref[...] = lax.dot_general(
                x_ref[b], wq, (((1,), (0,)), ((), ())),
                preferred_element_type=jnp.float32,
            ).astype(jnp.bfloat16)
            for h in range(HQ_PER):
                cols = slice(h * DH, (h + 1) * DH)
                k_h = kblk[buf, :, cols]
                v_h = vblk[buf, :, cols]
                for lo, hi, nk in ((0, SQ // 2, SKV // 2),
                                   (SQ // 2, SQ, SKV)):
                    scores = lax.dot_general(
                        q_ref[lo:hi, h * DH:(h + 1) * DH], k_h[:nk],
                        (((1,), (1,)), ((), ())),
                        preferred_element_type=jnp.float32,
                    )
                    e = jnp.exp(
                        scores * 0.125 + bias_ref[lo:hi, :nk]
                    ).astype(jnp.bfloat16)
                    s = lax.dot_general(
                        e, v_h[:nk], (((1,), (0,)), ((), ())),
                        preferred_element_type=jnp.float32,
                    )
                    denom = jnp.sum(
                        e.astype(jnp.float32), axis=-1, keepdims=True
                    )
                    ctx_ref[lo:hi, h * DH:(h + 1) * DH] = (
                        s / denom
                    ).astype(jnp.bfloat16)
            part = lax.dot_general(
                ctx_ref[...], woT, (((1,), (1,)), ((), ())),
                preferred_element_type=jnp.float32,
            )
            if slot == 0:
                out_ref[b] = part
            else:
                out_ref[b] = out_ref[b] + part

        for phase in range(N_PHASE):
            slot, b = divmod(phase, B_PER)
            buf = phase % 2
            if phase == 2:
                rdma_a.wait_recv()
                rdma_b.wait_recv()
                rdma_c.start()
                rdma_d.start()
            if phase == 6:
                rdma_c.wait_recv()
                rdma_d.wait_recv()
            nxt = issue_blk(phase + 1, 1 - buf) if phase < N_PHASE - 1 else []
            for d in pending:
                d.wait()
            compute_batch(slot, b, buf)
            pending = nxt

        rdma_a.wait_send()
        rdma_b.wait_send()
        rdma_c.wait_send()
        rdma_d.wait_send()

    return pl.pallas_call(
        body,
        out_shape=jax.ShapeDtypeStruct((B_PER, SQ, DMODEL), jnp.float32),
        in_specs=[pl.BlockSpec(memory_space=pltpu.VMEM)] * 4,
        out_specs=pl.BlockSpec(memory_space=pltpu.VMEM),
        scratch_shapes=[
            pltpu.VMEM((N_DEV, 2, DMODEL, HDIM), bf16),
            pltpu.VMEM((SQ, HDIM), bf16),
            pltpu.VMEM((SQ, HDIM), bf16),
            pltpu.VMEM((SQ, SKV), jnp.float32),
            pltpu.VMEM((2, SKV, HDIM), bf16),
            pltpu.VMEM((2, SKV, HDIM), bf16),
            pltpu.SemaphoreType.DMA((4,)),
            pltpu.SemaphoreType.DMA((4,)),
            pltpu.SemaphoreType.DMA((2,)),
        ],
        compiler_params=_CompilerParams(collective_id=0),
    )(x_bf, K_ord, V_ord, payload)


# device time: 60042 ns/iter; 2.8251x vs baseline; 1.0365x over previous
import jax
import jax.numpy as jnp
from jax import lax
from jax.experimental import pallas as pl
from jax.experimental.pallas import tpu as pltpu

N_DEV = 4
B_PER = 2
HQ_PER = 8
SQ = 512
SKV = 512
DH = 64
DMODEL = 768
HDIM = HQ_PER * DH
KVW = 32 * DH
BLK = 64
N_PHASE = N_DEV * B_PER

_CompilerParams = getattr(pltpu, "CompilerParams", None) or getattr(
    pltpu, "TPUCompilerParams"
)


def kernel(x, Wq, K_ext, V_ext, Wo):
    bf16 = jnp.bfloat16
    my = lax.axis_index("i")
    x_bf = x.astype(bf16)

    def prep(t):
        return lax.dynamic_slice_in_dim(
            t.reshape(8, SKV, KVW), my * B_PER, B_PER, axis=0
        ).astype(bf16)

    K_ord = prep(K_ext)
    V_ord = prep(V_ext)
    Wq_bf = Wq.astype(bf16)
    Wo_bf = Wo.astype(bf16)

    def body(
        x_ref, k_ref, v_ref, wq_ref, wo_ref, out_ref,
        comm_wq, comm_wo, q_ref, ctx_ref, bias_ref, kblk, vblk,
        send_sems, recv_sems, blk_sems,
    ):
        my_i = lax.axis_index("i")
        left = lax.rem(my_i + N_DEV - 1, N_DEV)
        right = lax.rem(my_i + 1, N_DEV)
        origins = [my_i, left, right, lax.rem(my_i + 2, N_DEV)]

        def issue_blk(phase, buf):
            slot, b = divmod(phase, B_PER)
            off = origins[slot] * HDIM
            dmas = []
            for src, dst in ((k_ref, kblk), (v_ref, vblk)):
                d = pltpu.make_async_copy(
                    src.at[b, :, pl.ds(off, HDIM)],
                    dst.at[buf],
                    blk_sems.at[buf],
                )
                d.start()
                dmas.append(d)
            return dmas

        pending = issue_blk(0, 0)

        row = lax.broadcasted_iota(jnp.int32, (SQ, SKV), 0)
        col = lax.broadcasted_iota(jnp.int32, (SQ, SKV), 1)
        bias_ref[...] = jnp.where(
            col // BLK <= row // BLK, 0.0, -1e9
        ).astype(jnp.float32)

        comm_wq[0] = wq_ref[...]
        comm_wo[0] = wo_ref[...]

        barrier_sem = pltpu.get_barrier_semaphore()
        for nbr in (left, right):
            pl.semaphore_signal(
                barrier_sem, inc=1,
                device_id=(nbr,), device_id_type=pl.DeviceIdType.MESH,
            )
        pl.semaphore_wait(barrier_sem, 2)

        def remote_copy(comm, src_slot, dst_slot, sem_idx, dev):
            return pltpu.make_async_remote_copy(
                src_ref=comm.at[src_slot], dst_ref=comm.at[dst_slot],
                send_sem=send_sems.at[sem_idx],
                recv_sem=recv_sems.at[sem_idx],
                device_id=(dev,),
                device_id_type=pl.DeviceIdType.MESH,
            )

        rdma_a_wq = remote_copy(comm_wq, 0, 1, 0, right)
        rdma_a_wo = remote_copy(comm_wo, 0, 1, 1, right)
        rdma_b_wq = remote_copy(comm_wq, 0, 2, 2, left)
        rdma_b_wo = remote_copy(comm_wo, 0, 2, 3, left)
        rdma_a_wq.start()
        rdma_b_wq.start()
        rdma_a_wo.start()
        rdma_b_wo.start()
        rdma_c_wq = remote_copy(comm_wq, 1, 3, 4, right)
        rdma_d_wo = remote_copy(comm_wo, 2, 3, 5, left)

        def compute_batch(slot, b, buf, wait_wo=None):
            wq = comm_wq[slot]
            q_ref[...] = lax.dot_general(
                x_ref[b], wq, (((1,), (0,)), ((), ())),
                preferred_element_type=jnp.float32,
            ).astype(jnp.bfloat16)
            for h in range(HQ_PER):
                cols = slice(h * DH, (h + 1) * DH)
                k_h = kblk[buf, :, cols]
                v_h = vblk[buf, :, cols]
                for lo, hi, nk in ((0, SQ // 2, SKV // 2),
                                   (SQ // 2, SQ, SKV)):
                    scores = lax.dot_general(
                        q_ref[lo:hi, h * DH:(h + 1) * DH], k_h[:nk],
                        (((1,), (1,)), ((), ())),
                        preferred_element_type=jnp.float32,
                    )
                    e = jnp.exp(
                        scores * 0.125 + bias_ref[lo:hi, :nk]
                    ).astype(jnp.bfloat16)
                    s = lax.dot_general(
                        e, v_h[:nk], (((1,), (0,)), ((), ())),
                        preferred_element_type=jnp.float32,
                    )
                    denom = jnp.sum(
                        e.astype(jnp.float32), axis=-1, keepdims=True
                    )
                    ctx_ref[lo:hi, h * DH:(h + 1) * DH] = (
                        s / denom
                    ).astype(jnp.bfloat16)
            if wait_wo is not None:
                wait_wo.wait_recv()
            part = lax.dot_general(
                ctx_ref[...], comm_wo[slot], (((1,), (0,)), ((), ())),
                preferred_element_type=jnp.float32,
            )
            if slot == 0:
                out_ref[b] = part
            else:
                out_ref[b] = out_ref[b] + part

        for phase in range(N_PHASE):
            slot, b = divmod(phase, B_PER)
            buf = phase % 2
            wait_wo = None
            if phase == 2:
                rdma_a_wq.wait_recv()
                rdma_c_wq.start()
                wait_wo = rdma_a_wo
            if phase == 4:
                rdma_b_wq.wait_recv()
                rdma_b_wo.wait_recv()
                rdma_d_wo.start()
            if phase == 6:
                rdma_c_wq.wait_recv()
                rdma_d_wo.wait_recv()
            nxt = issue_blk(phase + 1, 1 - buf) if phase < N_PHASE - 1 else []
            for d in pending:
                d.wait()
            compute_batch(slot, b, buf, wait_wo)
            pending = nxt

        rdma_a_wq.wait_send()
        rdma_a_wo.wait_send()
        rdma_b_wq.wait_send()
        rdma_b_wo.wait_send()
        rdma_c_wq.wait_send()
        rdma_d_wo.wait_send()

    return pl.pallas_call(
        body,
        out_shape=jax.ShapeDtypeStruct((B_PER, SQ, DMODEL), jnp.float32),
        in_specs=[pl.BlockSpec(memory_space=pltpu.VMEM)] * 5,
        out_specs=pl.BlockSpec(memory_space=pltpu.VMEM),
        scratch_shapes=[
            pltpu.VMEM((N_DEV, DMODEL, HDIM), bf16),
            pltpu.VMEM((N_DEV, HDIM, DMODEL), bf16),
            pltpu.VMEM((SQ, HDIM), bf16),
            pltpu.VMEM((SQ, HDIM), bf16),
            pltpu.VMEM((SQ, SKV), jnp.float32),
            pltpu.VMEM((2, SKV, HDIM), bf16),
            pltpu.VMEM((2, SKV, HDIM), bf16),
            pltpu.SemaphoreType.DMA((6,)),
            pltpu.SemaphoreType.DMA((6,)),
            pltpu.SemaphoreType.DMA((2,)),
        ],
        compiler_params=_CompilerParams(collective_id=0),
    )(x_bf, K_ord, V_ord, Wq_bf, Wo_bf)
